# baseline (speedup 1.0000x reference)
import jax
import jax.numpy as jnp
from jax.experimental import pallas as pl
from jax.experimental.pallas import tpu as pltpu

H = 256
W = 256
N = 4096
RB = 8
RG = 4
GB = 8
T_CULL = 23.0


def _project(pt_ref, o_ref):
    x = pt_ref[0:1, :]
    y = pt_ref[1:2, :]
    l1 = pt_ref[2:3, :] + 0.5
    l2 = pt_ref[3:4, :]
    l3 = pt_ref[4:5, :] + 0.5
    op = pt_ref[5:6, :]
    gx = (jnp.tanh(x) + 1.0) * (0.5 * W)
    gy = (jnp.tanh(y) + 1.0) * (0.5 * H)
    a = l1 * l1
    b = l1 * l2
    c = l2 * l2 + l3 * l3
    inv = 1.0 / (a * c - b * b)
    o_ref[0:1, :] = gx
    o_ref[1:2, :] = gy
    o_ref[2:3, :] = (-0.5) * c * inv
    o_ref[3:4, :] = (-0.5) * a * inv
    o_ref[4:5, :] = b * inv
    o_ref[5:6, :] = op * pt_ref[6:7, :]
    o_ref[6:7, :] = op * pt_ref[7:8, :]
    o_ref[7:8, :] = op * pt_ref[8:9, :]


def _raster(b_ref, p_ref, o_ref):
    band = pl.program_id(0)
    lo8 = b_ref[0, band]
    nch = b_ref[1, band]

    lane = jax.lax.broadcasted_iota(jnp.int32, (GB, 128), 1).astype(jnp.float32)
    px = [lane + 0.5, lane + 128.5]
    yb = (band * RB).astype(jnp.float32)

    for grp in range(RB // RG):
        def chunk_body(i, accs):
            base = lo8 + i * GB
            q = p_ref[pl.ds(base, GB), :]
            gx = jnp.broadcast_to(q[:, 0:1], (GB, 128))
            gy = jnp.broadcast_to(q[:, 1:2], (GB, 128))
            A = jnp.broadcast_to(q[:, 2:3], (GB, 128))
            D = jnp.broadcast_to(q[:, 3:4], (GB, 128))
            E = jnp.broadcast_to(q[:, 4:5], (GB, 128))
            col = [jnp.broadcast_to(q[:, 5 + ch:6 + ch], (GB, 128)) for ch in range(3)]

            out = list(accs)
            for r in range(RG):
                py = yb + (grp * RG + r + 0.5)
                dy = py - gy
                t1 = E * dy
                t2 = D * (dy * dy)
                for h in range(2):
                    dx = px[h] - gx
                    pw = (A * dx + t1) * dx + t2
                    e = jnp.exp(pw)
                    for ch in range(3):
                        k = (r * 2 + h) * 3 + ch
                        out[k] = out[k] + e * col[ch]
            return tuple(out)

        zero = jnp.zeros((GB, 128), dtype=jnp.float32)
        accs0 = tuple(zero for _ in range(RG * 2 * 3))
        accs = jax.lax.fori_loop(0, nch, chunk_body, accs0)
        for r in range(RG):
            for h in range(2):
                for ch in range(3):
                    v = jnp.sum(accs[(r * 2 + h) * 3 + ch], axis=0)
                    o_ref[ch, grp * RG + r, pl.ds(h * 128, 128)] = jnp.clip(v, 0.0, 1.0)


def kernel(xyz, cholesky, opacity, features_dc):
    l1 = cholesky[:, 0] + 0.5
    l2 = cholesky[:, 1]
    l3 = cholesky[:, 2] + 0.5
    rad = jnp.sqrt(2.0 * T_CULL * (l1 * l1 + l2 * l2 + l3 * l3))
    rmax = jnp.max(rad)
    cy = (jnp.tanh(xyz[:, 1]) + 1.0) * (0.5 * H)
    order = jnp.argsort(cy)
    cys = cy[order]
    p = jnp.concatenate([xyz, cholesky, opacity, features_dc], axis=1)[order]
    pt = jnp.pad(p, ((0, 0), (0, 7))).T

    proj_t = pl.pallas_call(
        _project,
        in_specs=[pl.BlockSpec((16, N), lambda: (0, 0))],
        out_specs=pl.BlockSpec((8, N), lambda: (0, 0)),
        out_shape=jax.ShapeDtypeStruct((8, N), jnp.float32),
    )(pt)
    proj = proj_t.T

    y0 = jnp.arange(H // RB, dtype=jnp.float32) * RB
    lo = jnp.searchsorted(cys, y0 - rmax, side="left").astype(jnp.int32)
    hi = jnp.searchsorted(cys, y0 + RB + rmax, side="right").astype(jnp.int32)
    lo8 = (lo // GB) * GB
    nch = (hi - lo8 + GB - 1) // GB
    binfo = jnp.stack([lo8, nch], axis=0)

    img = pl.pallas_call(
        _raster,
        grid=(H // RB,),
        in_specs=[
            pl.BlockSpec(memory_space=pltpu.SMEM),
            pl.BlockSpec((N, 8), lambda i: (0, 0)),
        ],
        out_specs=pl.BlockSpec((3, RB, W), lambda i: (0, i, 0)),
        out_shape=jax.ShapeDtypeStruct((3, H, W), jnp.float32),
        compiler_params=pltpu.CompilerParams(dimension_semantics=("parallel",)),
    )(binfo, proj)
    return img[None]


# speedup vs baseline: 1.1454x; 1.1454x over previous
import jax
import jax.numpy as jnp
from jax.experimental import pallas as pl
from jax.experimental.pallas import tpu as pltpu

H = 256
W = 256
N = 4096
RB = 8
RG = 4
GB = 8
T_CULL = 23.0


def _project(pt_ref, o_ref):
    x = pt_ref[0:1, :]
    y = pt_ref[1:2, :]
    l1 = pt_ref[2:3, :] + 0.5
    l2 = pt_ref[3:4, :]
    l3 = pt_ref[4:5, :] + 0.5
    op = pt_ref[5:6, :]
    gx = (jnp.tanh(x) + 1.0) * (0.5 * W)
    gy = (jnp.tanh(y) + 1.0) * (0.5 * H)
    a = l1 * l1
    b = l1 * l2
    c = l2 * l2 + l3 * l3
    inv = 1.0 / (a * c - b * b)
    lg2e = 1.4426950408889634
    o_ref[0:1, :] = gx
    o_ref[1:2, :] = gy
    o_ref[2:3, :] = (-0.5 * lg2e) * c * inv
    o_ref[3:4, :] = (-0.5 * lg2e) * a * inv
    o_ref[4:5, :] = lg2e * b * inv
    o_ref[5:6, :] = op * pt_ref[6:7, :]
    o_ref[6:7, :] = op * pt_ref[7:8, :]
    o_ref[7:8, :] = op * pt_ref[8:9, :]


def _raster(b_ref, p_ref, o_ref):
    band = pl.program_id(0)
    lo8 = b_ref[0, band]
    nch = b_ref[1, band]

    lane = jax.lax.broadcasted_iota(jnp.int32, (GB, 128), 1).astype(jnp.float32)
    px = [lane + 0.5, lane + 128.5]
    yb = (band * RB).astype(jnp.float32)

    def load_bcast(base):
        q = p_ref[pl.ds(base, GB), :]
        return tuple(jnp.broadcast_to(q[:, j:j + 1], (GB, 128)) for j in range(8))

    for grp in range(RB // RG):
        def chunk_body(i, carry):
            prm, accs = carry
            nxt = load_bcast(lo8 + (i + 1) * GB)
            gx, gy, A, D, E = prm[0], prm[1], prm[2], prm[3], prm[4]
            col = prm[5:8]

            out = list(accs)
            for r in range(RG):
                py = yb + (grp * RG + r + 0.5)
                dy = py - gy
                t1 = E * dy
                t2 = D * (dy * dy)
                for h in range(2):
                    dx = px[h] - gx
                    pw = (A * dx + t1) * dx + t2
                    e = jnp.exp2(pw)
                    for ch in range(3):
                        k = (r * 2 + h) * 3 + ch
                        out[k] = out[k] + e * col[ch]
            return nxt, tuple(out)

        zero = jnp.zeros((GB, 128), dtype=jnp.float32)
        accs0 = tuple(zero for _ in range(RG * 2 * 3))
        _, accs = jax.lax.fori_loop(0, nch, chunk_body, (load_bcast(lo8), accs0))
        for r in range(RG):
            for h in range(2):
                for ch in range(3):
                    v = jnp.sum(accs[(r * 2 + h) * 3 + ch], axis=0)
                    o_ref[ch, grp * RG + r, pl.ds(h * 128, 128)] = jnp.clip(v, 0.0, 1.0)


def kernel(xyz, cholesky, opacity, features_dc):
    l1 = cholesky[:, 0] + 0.5
    l2 = cholesky[:, 1]
    l3 = cholesky[:, 2] + 0.5
    rad = jnp.sqrt(2.0 * T_CULL * (l1 * l1 + l2 * l2 + l3 * l3))
    rmax = jnp.max(rad)
    cy = (jnp.tanh(xyz[:, 1]) + 1.0) * (0.5 * H)
    order = jnp.argsort(cy)
    cys = cy[order]
    p = jnp.concatenate([xyz, cholesky, opacity, features_dc], axis=1)[order]
    pt = jnp.pad(p, ((0, 0), (0, 7))).T

    proj_t = pl.pallas_call(
        _project,
        in_specs=[pl.BlockSpec((16, N), lambda: (0, 0))],
        out_specs=pl.BlockSpec((8, N), lambda: (0, 0)),
        out_shape=jax.ShapeDtypeStruct((8, N), jnp.float32),
    )(pt)
    proj = jnp.pad(proj_t.T, ((0, GB), (0, 0)))

    y0 = jnp.arange(H // RB, dtype=jnp.float32) * RB
    lo = jnp.searchsorted(cys, y0 - rmax, side="left").astype(jnp.int32)
    hi = jnp.searchsorted(cys, y0 + RB + rmax, side="right").astype(jnp.int32)
    lo8 = (lo // GB) * GB
    nch = (hi - lo8 + GB - 1) // GB
    binfo = jnp.stack([lo8, nch], axis=0)

    img = pl.pallas_call(
        _raster,
        grid=(H // RB,),
        in_specs=[
            pl.BlockSpec(memory_space=pltpu.SMEM),
            pl.BlockSpec((N + GB, 8), lambda i: (0, 0)),
        ],
        out_specs=pl.BlockSpec((3, RB, W), lambda i: (0, i, 0)),
        out_shape=jax.ShapeDtypeStruct((3, H, W), jnp.float32),
        compiler_params=pltpu.CompilerParams(dimension_semantics=("parallel",)),
    )(binfo, proj)
    return img[None]


# speedup vs baseline: 2.8567x; 2.4941x over previous
import jax
import jax.numpy as jnp
from jax.experimental import pallas as pl
from jax.experimental.pallas import tpu as pltpu

H = 256
W = 256
N = 4096
RB = 8
GB = 8
KC = 8
NP = N + KC * GB
T_CULL = 23.0


def _project(pt_ref, o_ref):
    x = pt_ref[0:1, :]
    y = pt_ref[1:2, :]
    l1 = pt_ref[2:3, :] + 0.5
    l2 = pt_ref[3:4, :]
    l3 = pt_ref[4:5, :] + 0.5
    op = pt_ref[5:6, :]
    gx = (jnp.tanh(x) + 1.0) * (0.5 * W)
    gy = (jnp.tanh(y) + 1.0) * (0.5 * H)
    a = l1 * l1
    b = l1 * l2
    c = l2 * l2 + l3 * l3
    inv = 1.0 / (a * c - b * b)
    lg2e = 1.4426950408889634
    o_ref[0:1, :] = gx
    o_ref[1:2, :] = gy
    o_ref[2:3, :] = (-0.5 * lg2e) * c * inv
    o_ref[3:4, :] = (-0.5 * lg2e) * a * inv
    o_ref[4:5, :] = lg2e * b * inv
    o_ref[5:6, :] = op * pt_ref[6:7, :]
    o_ref[6:7, :] = op * pt_ref[7:8, :]
    o_ref[7:8, :] = op * pt_ref[8:9, :]


def _raster(b_ref, p_ref, o_ref, s_ref):
    band = pl.program_id(0)
    lo8 = b_ref[0, band]
    nbl = b_ref[1, band]

    lane = jax.lax.broadcasted_iota(jnp.int32, (GB, 128), 1).astype(jnp.float32)
    px = [lane + 0.5, lane + 128.5]
    yb = (band * RB).astype(jnp.float32)
    zero = jnp.zeros((GB, 128), dtype=jnp.float32)

    for k in range(RB * 2 * 3):
        s_ref[k] = zero

    RG = 4

    def block_body(ib, _):
        base0 = lo8 + ib * (KC * GB)
        for grp in range(RB // RG):
            accs = [zero] * (RG * 2 * 3)
            for j in range(KC):
                q = p_ref[pl.ds(base0 + j * GB, GB), :]
                gx = jnp.broadcast_to(q[:, 0:1], (GB, 128))
                gy = jnp.broadcast_to(q[:, 1:2], (GB, 128))
                A = jnp.broadcast_to(q[:, 2:3], (GB, 128))
                D = jnp.broadcast_to(q[:, 3:4], (GB, 128))
                E = jnp.broadcast_to(q[:, 4:5], (GB, 128))
                col = [jnp.broadcast_to(q[:, 5 + ch:6 + ch], (GB, 128)) for ch in range(3)]
                for r in range(RG):
                    py = yb + (grp * RG + r + 0.5)
                    dy = py - gy
                    t1 = E * dy
                    t2 = D * (dy * dy)
                    for h in range(2):
                        dx = px[h] - gx
                        pw = (A * dx + t1) * dx + t2
                        e = jnp.exp2(pw)
                        for ch in range(3):
                            k = (r * 2 + h) * 3 + ch
                            accs[k] = accs[k] + e * col[ch]
            for k in range(RG * 2 * 3):
                ks = grp * (RG * 2 * 3) + k
                s_ref[ks] = s_ref[ks] + accs[k]
        return 0

    jax.lax.fori_loop(0, nbl, block_body, 0)
    for r in range(RB):
        for h in range(2):
            for ch in range(3):
                ks = (r // 4) * 24 + ((r % 4) * 2 + h) * 3 + ch
                v = jnp.sum(s_ref[ks], axis=0)
                o_ref[ch, r, pl.ds(h * 128, 128)] = jnp.clip(v, 0.0, 1.0)


def kernel(xyz, cholesky, opacity, features_dc):
    l1 = cholesky[:, 0] + 0.5
    l2 = cholesky[:, 1]
    l3 = cholesky[:, 2] + 0.5
    rad = jnp.sqrt(2.0 * T_CULL * (l1 * l1 + l2 * l2 + l3 * l3))
    rmax = jnp.max(rad)
    cy = (jnp.tanh(xyz[:, 1]) + 1.0) * (0.5 * H)
    order = jnp.argsort(cy)
    cys = cy[order]
    p = jnp.concatenate([xyz, cholesky, opacity, features_dc], axis=1)[order]
    pt = jnp.pad(p, ((0, 0), (0, 7))).T

    proj_t = pl.pallas_call(
        _project,
        in_specs=[pl.BlockSpec((16, N), lambda: (0, 0))],
        out_specs=pl.BlockSpec((8, N), lambda: (0, 0)),
        out_shape=jax.ShapeDtypeStruct((8, N), jnp.float32),
    )(pt)
    proj = jnp.pad(proj_t.T, ((0, NP - N), (0, 0)))

    y0 = jnp.arange(H // RB, dtype=jnp.float32) * RB
    lo = jnp.searchsorted(cys, y0 - rmax, side="left").astype(jnp.int32)
    hi = jnp.searchsorted(cys, y0 + RB + rmax, side="right").astype(jnp.int32)
    lo8 = (lo // GB) * GB
    nch = (hi - lo8 + GB - 1) // GB
    nbl = (nch + KC - 1) // KC
    binfo = jnp.stack([lo8, nbl], axis=0)

    img = pl.pallas_call(
        _raster,
        grid=(H // RB,),
        in_specs=[
            pl.BlockSpec(memory_space=pltpu.SMEM),
            pl.BlockSpec((NP, 8), lambda i: (0, 0)),
        ],
        out_specs=pl.BlockSpec((3, RB, W), lambda i: (0, i, 0)),
        out_shape=jax.ShapeDtypeStruct((3, H, W), jnp.float32),
        scratch_shapes=[pltpu.VMEM((RB * 2 * 3, GB, 128), jnp.float32)],
        compiler_params=pltpu.CompilerParams(dimension_semantics=("parallel",)),
    )(binfo, proj)
    return img[None]


# speedup vs baseline: 3.8918x; 1.3623x over previous
import jax
import jax.numpy as jnp
from jax.experimental import pallas as pl
from jax.experimental.pallas import tpu as pltpu

H = 256
W = 256
N = 4096
RB = 8
GB = 8
KC = 16
NP = N + KC * GB
T_CULL = 23.0


def _project(pt_ref, o_ref):
    x = pt_ref[0:1, :]
    y = pt_ref[1:2, :]
    l1 = pt_ref[2:3, :] + 0.5
    l2 = pt_ref[3:4, :]
    l3 = pt_ref[4:5, :] + 0.5
    op = pt_ref[5:6, :]
    gx = (jnp.tanh(x) + 1.0) * (0.5 * W)
    gy = (jnp.tanh(y) + 1.0) * (0.5 * H)
    a = l1 * l1
    b = l1 * l2
    c = l2 * l2 + l3 * l3
    inv = 1.0 / (a * c - b * b)
    lg2e = 1.4426950408889634
    o_ref[0:1, :] = gx
    o_ref[1:2, :] = gy
    o_ref[2:3, :] = (-0.5 * lg2e) * c * inv
    o_ref[3:4, :] = (-0.5 * lg2e) * a * inv
    o_ref[4:5, :] = lg2e * b * inv
    o_ref[5:6, :] = op * pt_ref[6:7, :]
    o_ref[6:7, :] = op * pt_ref[7:8, :]
    o_ref[7:8, :] = op * pt_ref[8:9, :]


def _raster(b_ref, p_ref, o_ref, s_ref):
    band = pl.program_id(0)
    lo8 = b_ref[0, band]
    nbl = b_ref[1, band]

    lane = jax.lax.broadcasted_iota(jnp.int32, (GB, 128), 1).astype(jnp.float32)
    px = [lane + 0.5, lane + 128.5]
    yb = (band * RB).astype(jnp.float32)
    zero = jnp.zeros((GB, 128), dtype=jnp.float32)

    for k in range(RB * 2 * 3):
        s_ref[k] = zero

    RG = 4

    def block_body(ib, _):
        base0 = lo8 + ib * (KC * GB)
        for grp in range(RB // RG):
            accs = [zero] * (RG * 2 * 3)
            for j in range(KC):
                q = p_ref[pl.ds(base0 + j * GB, GB), :]
                gx = jnp.broadcast_to(q[:, 0:1], (GB, 128))
                gy = jnp.broadcast_to(q[:, 1:2], (GB, 128))
                A = jnp.broadcast_to(q[:, 2:3], (GB, 128))
                D = jnp.broadcast_to(q[:, 3:4], (GB, 128))
                E = jnp.broadcast_to(q[:, 4:5], (GB, 128))
                col = [jnp.broadcast_to(q[:, 5 + ch:6 + ch], (GB, 128)) for ch in range(3)]
                for r in range(RG):
                    py = yb + (grp * RG + r + 0.5)
                    dy = py - gy
                    t1 = E * dy
                    t2 = D * (dy * dy)
                    for h in range(2):
                        dx = px[h] - gx
                        pw = (A * dx + t1) * dx + t2
                        e = jnp.exp2(pw)
                        for ch in range(3):
                            k = (r * 2 + h) * 3 + ch
                            accs[k] = accs[k] + e * col[ch]
            for k in range(RG * 2 * 3):
                ks = grp * (RG * 2 * 3) + k
                s_ref[ks] = s_ref[ks] + accs[k]
        return 0

    jax.lax.fori_loop(0, nbl, block_body, 0)
    for r in range(RB):
        for h in range(2):
            for ch in range(3):
                ks = (r // 4) * 24 + ((r % 4) * 2 + h) * 3 + ch
                v = jnp.sum(s_ref[ks], axis=0)
                o_ref[ch, r, pl.ds(h * 128, 128)] = jnp.clip(v, 0.0, 1.0)


def kernel(xyz, cholesky, opacity, features_dc):
    l1 = cholesky[:, 0] + 0.5
    l2 = cholesky[:, 1]
    l3 = cholesky[:, 2] + 0.5
    rad = jnp.sqrt(2.0 * T_CULL * (l1 * l1 + l2 * l2 + l3 * l3))
    rmax = jnp.max(rad)
    order = jnp.argsort(xyz[:, 1])
    p = jnp.concatenate([xyz, cholesky, opacity, features_dc], axis=1)[order]
    cys = (jnp.tanh(p[:, 1]) + 1.0) * (0.5 * H)
    pt = jnp.pad(p, ((0, 0), (0, 7))).T

    proj_t = pl.pallas_call(
        _project,
        in_specs=[pl.BlockSpec((16, N), lambda: (0, 0))],
        out_specs=pl.BlockSpec((8, N), lambda: (0, 0)),
        out_shape=jax.ShapeDtypeStruct((8, N), jnp.float32),
    )(pt)
    proj = jnp.pad(proj_t.T, ((0, NP - N), (0, 0)))

    y0 = jnp.arange(H // RB, dtype=jnp.float32) * RB
    lo = jnp.sum(cys[None, :] < (y0 - rmax)[:, None], axis=1).astype(jnp.int32)
    hi = jnp.sum(cys[None, :] <= (y0 + RB + rmax)[:, None], axis=1).astype(jnp.int32)
    lo8 = (lo // GB) * GB
    nch = (hi - lo8 + GB - 1) // GB
    nbl = (nch + KC - 1) // KC
    binfo = jnp.stack([lo8, nbl], axis=0)

    img = pl.pallas_call(
        _raster,
        grid=(H // RB,),
        in_specs=[
            pl.BlockSpec(memory_space=pltpu.SMEM),
            pl.BlockSpec((NP, 8), lambda i: (0, 0)),
        ],
        out_specs=pl.BlockSpec((3, RB, W), lambda i: (0, i, 0)),
        out_shape=jax.ShapeDtypeStruct((3, H, W), jnp.float32),
        scratch_shapes=[pltpu.VMEM((RB * 2 * 3, GB, 128), jnp.float32)],
        compiler_params=pltpu.CompilerParams(dimension_semantics=("parallel",)),
    )(binfo, proj)
    return img[None]


# speedup vs baseline: 3.9345x; 1.0110x over previous
import functools

import jax
import jax.numpy as jnp
from jax.experimental import pallas as pl
from jax.experimental.pallas import tpu as pltpu
from jax.experimental.pallas import tpu_sc as plsc

H = 256
W = 256
N = 4096
RB = 8
GB = 8
KC = 16
NP = N + KC * GB
T_CULL = 23.0


def _sc_gather(table, idx):
    info = plsc.get_sparse_core_info()
    nc, ns = info.num_cores, info.num_subcores
    nw = nc * ns
    b, d = table.shape
    b_per_w = b // nw
    mesh = plsc.VectorSubcoreMesh(core_axis_name="c", subcore_axis_name="s")

    @functools.partial(
        pl.kernel, mesh=mesh,
        out_type=jax.ShapeDtypeStruct((b, d), jnp.float32),
        scratch_types=[
            pltpu.VMEM((b_per_w,), jnp.int32),
            pltpu.VMEM((b_per_w, d), jnp.float32),
            pltpu.SemaphoreType.DMA,
        ],
    )
    def k(table_hbm, idx_hbm, out_hbm, idx_v, rows_v, sem):
        wid = jax.lax.axis_index("s") * nc + jax.lax.axis_index("c")
        base = wid * b_per_w
        pltpu.sync_copy(idx_hbm.at[pl.ds(base, b_per_w)], idx_v)
        pltpu.async_copy(table_hbm.at[idx_v], rows_v, sem).wait()
        pltpu.sync_copy(rows_v, out_hbm.at[pl.ds(base, b_per_w)])

    return k(table, idx)


def _project(pt_ref, o_ref):
    x = pt_ref[0:1, :]
    y = pt_ref[1:2, :]
    l1 = pt_ref[2:3, :] + 0.5
    l2 = pt_ref[3:4, :]
    l3 = pt_ref[4:5, :] + 0.5
    op = pt_ref[5:6, :]
    gx = (jnp.tanh(x) + 1.0) * (0.5 * W)
    gy = (jnp.tanh(y) + 1.0) * (0.5 * H)
    a = l1 * l1
    b = l1 * l2
    c = l2 * l2 + l3 * l3
    inv = 1.0 / (a * c - b * b)
    lg2e = 1.4426950408889634
    o_ref[0:1, :] = gx
    o_ref[1:2, :] = gy
    o_ref[2:3, :] = (-0.5 * lg2e) * c * inv
    o_ref[3:4, :] = (-0.5 * lg2e) * a * inv
    o_ref[4:5, :] = lg2e * b * inv
    o_ref[5:6, :] = op * pt_ref[6:7, :]
    o_ref[6:7, :] = op * pt_ref[7:8, :]
    o_ref[7:8, :] = op * pt_ref[8:9, :]


def _raster(b_ref, p_ref, o_ref, s_ref):
    band = pl.program_id(0)
    lo8 = b_ref[0, band]
    nbl = b_ref[1, band]

    lane = jax.lax.broadcasted_iota(jnp.int32, (GB, 128), 1).astype(jnp.float32)
    px = [lane + 0.5, lane + 128.5]
    yb = (band * RB).astype(jnp.float32)
    zero = jnp.zeros((GB, 128), dtype=jnp.float32)

    for k in range(RB * 2 * 3):
        s_ref[k] = zero

    RG = 4

    def block_body(ib, _):
        base0 = lo8 + ib * (KC * GB)
        for grp in range(RB // RG):
            accs = [zero] * (RG * 2 * 3)
            for j in range(KC):
                q = p_ref[pl.ds(base0 + j * GB, GB), :]
                gx = jnp.broadcast_to(q[:, 0:1], (GB, 128))
                gy = jnp.broadcast_to(q[:, 1:2], (GB, 128))
                A = jnp.broadcast_to(q[:, 2:3], (GB, 128))
                D = jnp.broadcast_to(q[:, 3:4], (GB, 128))
                E = jnp.broadcast_to(q[:, 4:5], (GB, 128))
                col = [jnp.broadcast_to(q[:, 5 + ch:6 + ch], (GB, 128)) for ch in range(3)]
                for r in range(RG):
                    py = yb + (grp * RG + r + 0.5)
                    dy = py - gy
                    t1 = E * dy
                    t2 = D * (dy * dy)
                    for h in range(2):
                        dx = px[h] - gx
                        pw = (A * dx + t1) * dx + t2
                        e = jnp.exp2(pw)
                        for ch in range(3):
                            k = (r * 2 + h) * 3 + ch
                            accs[k] = accs[k] + e * col[ch]
            for k in range(RG * 2 * 3):
                ks = grp * (RG * 2 * 3) + k
                s_ref[ks] = s_ref[ks] + accs[k]
        return 0

    jax.lax.fori_loop(0, nbl, block_body, 0)
    for r in range(RB):
        for h in range(2):
            for ch in range(3):
                ks = (r // 4) * 24 + ((r % 4) * 2 + h) * 3 + ch
                v = jnp.sum(s_ref[ks], axis=0)
                o_ref[ch, r, pl.ds(h * 128, 128)] = jnp.clip(v, 0.0, 1.0)


def kernel(xyz, cholesky, opacity, features_dc):
    l1 = cholesky[:, 0] + 0.5
    l2 = cholesky[:, 1]
    l3 = cholesky[:, 2] + 0.5
    rad = jnp.sqrt(2.0 * T_CULL * (l1 * l1 + l2 * l2 + l3 * l3))
    rmax = jnp.max(rad)
    order = jnp.argsort(xyz[:, 1]).astype(jnp.int32)
    table = jnp.pad(jnp.concatenate([xyz, cholesky, opacity, features_dc], axis=1),
                    ((0, 0), (0, 119)))
    p = _sc_gather(table, order)[:, :16]
    cys = (jnp.tanh(p[:, 1]) + 1.0) * (0.5 * H)
    pt = p.T

    proj_t = pl.pallas_call(
        _project,
        in_specs=[pl.BlockSpec((16, N), lambda: (0, 0))],
        out_specs=pl.BlockSpec((8, N), lambda: (0, 0)),
        out_shape=jax.ShapeDtypeStruct((8, N), jnp.float32),
    )(pt)
    proj = jnp.pad(proj_t.T, ((0, NP - N), (0, 0)))

    y0 = jnp.arange(H // RB, dtype=jnp.float32) * RB
    lo = jnp.sum(cys[None, :] < (y0 - rmax)[:, None], axis=1).astype(jnp.int32)
    hi = jnp.sum(cys[None, :] <= (y0 + RB + rmax)[:, None], axis=1).astype(jnp.int32)
    lo8 = (lo // GB) * GB
    nch = (hi - lo8 + GB - 1) // GB
    nbl = (nch + KC - 1) // KC
    binfo = jnp.stack([lo8, nbl], axis=0)

    img = pl.pallas_call(
        _raster,
        grid=(H // RB,),
        in_specs=[
            pl.BlockSpec(memory_space=pltpu.SMEM),
            pl.BlockSpec((NP, 8), lambda i: (0, 0)),
        ],
        out_specs=pl.BlockSpec((3, RB, W), lambda i: (0, i, 0)),
        out_shape=jax.ShapeDtypeStruct((3, H, W), jnp.float32),
        scratch_shapes=[pltpu.VMEM((RB * 2 * 3, GB, 128), jnp.float32)],
        compiler_params=pltpu.CompilerParams(dimension_semantics=("parallel",)),
    )(binfo, proj)
    return img[None]


# speedup vs baseline: 4.1693x; 1.0597x over previous
import functools

import jax
import jax.numpy as jnp
from jax.experimental import pallas as pl
from jax.experimental.pallas import tpu as pltpu
from jax.experimental.pallas import tpu_sc as plsc

H = 256
W = 256
N = 4096
RB = 8
GB = 8
KC = 16
NP = N + KC * GB
T_CULL = 16.0


def _sc_gather(table, idx):
    info = plsc.get_sparse_core_info()
    nc, ns = info.num_cores, info.num_subcores
    nw = nc * ns
    b, d = table.shape
    b_per_w = b // nw
    mesh = plsc.VectorSubcoreMesh(core_axis_name="c", subcore_axis_name="s")

    @functools.partial(
        pl.kernel, mesh=mesh,
        out_type=jax.ShapeDtypeStruct((b, d), jnp.float32),
        scratch_types=[
            pltpu.VMEM((b_per_w,), jnp.int32),
            pltpu.VMEM((b_per_w, d), jnp.float32),
            pltpu.SemaphoreType.DMA,
        ],
    )
    def k(table_hbm, idx_hbm, out_hbm, idx_v, rows_v, sem):
        wid = jax.lax.axis_index("s") * nc + jax.lax.axis_index("c")
        base = wid * b_per_w
        pltpu.sync_copy(idx_hbm.at[pl.ds(base, b_per_w)], idx_v)
        pltpu.async_copy(table_hbm.at[idx_v], rows_v, sem).wait()
        pltpu.sync_copy(rows_v, out_hbm.at[pl.ds(base, b_per_w)])

    return k(table, idx)


def _project(pt_ref, o_ref):
    x = pt_ref[0:1, :]
    y = pt_ref[1:2, :]
    l1 = pt_ref[2:3, :] + 0.5
    l2 = pt_ref[3:4, :]
    l3 = pt_ref[4:5, :] + 0.5
    op = pt_ref[5:6, :]
    gx = (jnp.tanh(x) + 1.0) * (0.5 * W)
    gy = (jnp.tanh(y) + 1.0) * (0.5 * H)
    a = l1 * l1
    b = l1 * l2
    c = l2 * l2 + l3 * l3
    inv = 1.0 / (a * c - b * b)
    lg2e = 1.4426950408889634
    o_ref[0:1, 0:N] = gx
    o_ref[1:2, 0:N] = gy
    o_ref[2:3, 0:N] = (-0.5 * lg2e) * c * inv
    o_ref[3:4, 0:N] = (-0.5 * lg2e) * a * inv
    o_ref[4:5, 0:N] = lg2e * b * inv
    o_ref[5:6, 0:N] = op * pt_ref[6:7, :]
    o_ref[6:7, 0:N] = op * pt_ref[7:8, :]
    o_ref[7:8, 0:N] = op * pt_ref[8:9, :]
    o_ref[:, N:NP] = jnp.zeros((8, NP - N), jnp.float32)


def _raster(b_ref, p_ref, o_ref, s_ref):
    band = pl.program_id(0)
    lo8 = b_ref[0, band]
    nbl = b_ref[1, band]

    lane = jax.lax.broadcasted_iota(jnp.int32, (GB, 128), 1).astype(jnp.float32)
    px = [lane + 0.5, lane + 128.5]
    yb = (band * RB).astype(jnp.float32)
    zero = jnp.zeros((GB, 128), dtype=jnp.float32)

    for k in range(RB * 2 * 3):
        s_ref[k] = zero

    RG = 4

    def block_body(ib, _):
        base0 = lo8 + ib * (KC * GB)
        for grp in range(RB // RG):
            accs = [zero] * (RG * 2 * 3)
            for j in range(KC):
                q = p_ref[pl.ds(base0 + j * GB, GB), :]
                gx = jnp.broadcast_to(q[:, 0:1], (GB, 128))
                gy = jnp.broadcast_to(q[:, 1:2], (GB, 128))
                A = jnp.broadcast_to(q[:, 2:3], (GB, 128))
                D = jnp.broadcast_to(q[:, 3:4], (GB, 128))
                E = jnp.broadcast_to(q[:, 4:5], (GB, 128))
                col = [jnp.broadcast_to(q[:, 5 + ch:6 + ch], (GB, 128)) for ch in range(3)]
                for r in range(RG):
                    py = yb + (grp * RG + r + 0.5)
                    dy = py - gy
                    t1 = E * dy
                    t2 = D * (dy * dy)
                    for h in range(2):
                        dx = px[h] - gx
                        pw = (A * dx + t1) * dx + t2
                        e = jnp.exp2(pw)
                        for ch in range(3):
                            k = (r * 2 + h) * 3 + ch
                            accs[k] = accs[k] + e * col[ch]
            for k in range(RG * 2 * 3):
                ks = grp * (RG * 2 * 3) + k
                s_ref[ks] = s_ref[ks] + accs[k]
        return 0

    jax.lax.fori_loop(0, nbl, block_body, 0)
    for r in range(RB):
        for h in range(2):
            for ch in range(3):
                ks = (r // 4) * 24 + ((r % 4) * 2 + h) * 3 + ch
                v = jnp.sum(s_ref[ks], axis=0)
                o_ref[ch, r, pl.ds(h * 128, 128)] = jnp.clip(v, 0.0, 1.0)


def kernel(xyz, cholesky, opacity, features_dc):
    l1 = cholesky[:, 0] + 0.5
    l2 = cholesky[:, 1]
    l3 = cholesky[:, 2] + 0.5
    rad = jnp.sqrt(2.0 * T_CULL * (l1 * l1 + l2 * l2 + l3 * l3))
    rmax = jnp.max(rad)
    order = jnp.argsort(xyz[:, 1]).astype(jnp.int32)
    table = jnp.pad(jnp.concatenate([xyz, cholesky, opacity, features_dc], axis=1),
                    ((0, 0), (0, 119)))
    p = _sc_gather(table, order)[:, :16]
    cys = (jnp.tanh(p[:, 1]) + 1.0) * (0.5 * H)
    pt = p.T

    proj_t = pl.pallas_call(
        _project,
        in_specs=[pl.BlockSpec((16, N), lambda: (0, 0))],
        out_specs=pl.BlockSpec((8, NP), lambda: (0, 0)),
        out_shape=jax.ShapeDtypeStruct((8, NP), jnp.float32),
    )(pt)
    proj = proj_t.T

    y0 = jnp.arange(H // RB, dtype=jnp.float32) * RB
    lo = jnp.sum(cys[None, :] < (y0 - rmax)[:, None], axis=1).astype(jnp.int32)
    hi = jnp.sum(cys[None, :] <= (y0 + RB + rmax)[:, None], axis=1).astype(jnp.int32)
    lo8 = (lo // GB) * GB
    nch = (hi - lo8 + GB - 1) // GB
    nbl = (nch + KC - 1) // KC
    binfo = jnp.stack([lo8, nbl], axis=0)

    img = pl.pallas_call(
        _raster,
        grid=(H // RB,),
        in_specs=[
            pl.BlockSpec(memory_space=pltpu.SMEM),
            pl.BlockSpec((NP, 8), lambda i: (0, 0)),
        ],
        out_specs=pl.BlockSpec((3, RB, W), lambda i: (0, i, 0)),
        out_shape=jax.ShapeDtypeStruct((3, H, W), jnp.float32),
        scratch_shapes=[pltpu.VMEM((RB * 2 * 3, GB, 128), jnp.float32)],
        compiler_params=pltpu.CompilerParams(dimension_semantics=("parallel",)),
    )(binfo, proj)
    return img[None]


# speedup vs baseline: 4.2240x; 1.0131x over previous
import functools

import jax
import jax.numpy as jnp
from jax.experimental import pallas as pl
from jax.experimental.pallas import tpu as pltpu
from jax.experimental.pallas import tpu_sc as plsc

H = 256
W = 256
N = 4096
RB = 8
GB = 8
KC = 16
NP = N + KC * GB
T_CULL = 16.0


def _sc_gather(table, idx):
    info = plsc.get_sparse_core_info()
    nc, ns = info.num_cores, info.num_subcores
    nw = nc * ns
    b, d = table.shape
    b_per_w = b // nw
    mesh = plsc.VectorSubcoreMesh(core_axis_name="c", subcore_axis_name="s")

    @functools.partial(
        pl.kernel, mesh=mesh,
        out_type=jax.ShapeDtypeStruct((b, d), jnp.float32),
        scratch_types=[
            pltpu.VMEM((b_per_w,), jnp.int32),
            pltpu.VMEM((b_per_w, d), jnp.float32),
            pltpu.SemaphoreType.DMA,
        ],
    )
    def k(table_hbm, idx_hbm, out_hbm, idx_v, rows_v, sem):
        wid = jax.lax.axis_index("s") * nc + jax.lax.axis_index("c")
        base = wid * b_per_w
        pltpu.sync_copy(idx_hbm.at[pl.ds(base, b_per_w)], idx_v)
        pltpu.async_copy(table_hbm.at[idx_v], rows_v, sem).wait()
        pltpu.sync_copy(rows_v, out_hbm.at[pl.ds(base, b_per_w)])

    return k(table, idx)


def _project(pt_ref, o_ref, b_ref):
    x = pt_ref[0:1, :]
    y = pt_ref[1:2, :]
    l1 = pt_ref[2:3, :] + 0.5
    l2 = pt_ref[3:4, :]
    l3 = pt_ref[4:5, :] + 0.5
    op = pt_ref[5:6, :]
    gx = (jnp.tanh(x) + 1.0) * (0.5 * W)
    gy = (jnp.tanh(y) + 1.0) * (0.5 * H)
    a = l1 * l1
    b = l1 * l2
    c = l2 * l2 + l3 * l3
    inv = 1.0 / (a * c - b * b)
    lg2e = 1.4426950408889634
    o_ref[0:1, 0:N] = gx
    o_ref[1:2, 0:N] = gy
    o_ref[2:3, 0:N] = (-0.5 * lg2e) * c * inv
    o_ref[3:4, 0:N] = (-0.5 * lg2e) * a * inv
    o_ref[4:5, 0:N] = lg2e * b * inv
    o_ref[5:6, 0:N] = op * pt_ref[6:7, :]
    o_ref[6:7, 0:N] = op * pt_ref[7:8, :]
    o_ref[7:8, 0:N] = op * pt_ref[8:9, :]
    o_ref[:, N:NP] = jnp.zeros((8, NP - N), jnp.float32)

    rmax = jnp.sqrt((2.0 * T_CULL) * jnp.max(a + c))
    y0 = jax.lax.broadcasted_iota(jnp.int32, (H // RB, 1), 0).astype(jnp.float32) * RB
    lo = jnp.sum((gy < (y0 - rmax)).astype(jnp.float32), axis=1, keepdims=True)
    hi = jnp.sum((gy <= (y0 + (RB + rmax))).astype(jnp.float32), axis=1, keepdims=True)
    lo8 = jnp.floor(lo * (1.0 / GB)) * GB
    nch = jnp.floor((hi - lo8 + (GB - 1)) * (1.0 / GB))
    nbl = jnp.floor((nch + (KC - 1)) * (1.0 / KC))
    pad = jnp.zeros((H // RB, 6), jnp.float32)
    b_ref[...] = jnp.concatenate([lo8, nbl, pad], axis=1).astype(jnp.int32)


def _raster(b_ref, p_ref, o_ref, s_ref):
    band = pl.program_id(0)
    lo8 = b_ref[band, 0]
    nbl = b_ref[band, 1]

    lane = jax.lax.broadcasted_iota(jnp.int32, (GB, 128), 1).astype(jnp.float32)
    px = [lane + 0.5, lane + 128.5]
    yb = (band * RB).astype(jnp.float32)
    zero = jnp.zeros((GB, 128), dtype=jnp.float32)

    for k in range(RB * 2 * 3):
        s_ref[k] = zero

    RG = 4

    def block_body(ib, _):
        base0 = lo8 + ib * (KC * GB)
        for grp in range(RB // RG):
            accs = [zero] * (RG * 2 * 3)
            for j in range(KC):
                q = p_ref[pl.ds(base0 + j * GB, GB), :]
                gx = jnp.broadcast_to(q[:, 0:1], (GB, 128))
                gy = jnp.broadcast_to(q[:, 1:2], (GB, 128))
                A = jnp.broadcast_to(q[:, 2:3], (GB, 128))
                D = jnp.broadcast_to(q[:, 3:4], (GB, 128))
                E = jnp.broadcast_to(q[:, 4:5], (GB, 128))
                col = [jnp.broadcast_to(q[:, 5 + ch:6 + ch], (GB, 128)) for ch in range(3)]
                for r in range(RG):
                    py = yb + (grp * RG + r + 0.5)
                    dy = py - gy
                    t1 = E * dy
                    t2 = D * (dy * dy)
                    for h in range(2):
                        dx = px[h] - gx
                        pw = (A * dx + t1) * dx + t2
                        e = jnp.exp2(pw)
                        for ch in range(3):
                            k = (r * 2 + h) * 3 + ch
                            accs[k] = accs[k] + e * col[ch]
            for k in range(RG * 2 * 3):
                ks = grp * (RG * 2 * 3) + k
                s_ref[ks] = s_ref[ks] + accs[k]
        return 0

    jax.lax.fori_loop(0, nbl, block_body, 0)
    for r in range(RB):
        for h in range(2):
            for ch in range(3):
                ks = (r // 4) * 24 + ((r % 4) * 2 + h) * 3 + ch
                v = jnp.sum(s_ref[ks], axis=0)
                o_ref[ch, r, pl.ds(h * 128, 128)] = jnp.clip(v, 0.0, 1.0)


def kernel(xyz, cholesky, opacity, features_dc):
    order = jnp.argsort(xyz[:, 1]).astype(jnp.int32)
    table = jnp.pad(jnp.concatenate([xyz, cholesky, opacity, features_dc], axis=1),
                    ((0, 0), (0, 119)))
    p = _sc_gather(table, order)[:, :16]
    pt = p.T

    proj_t, binfo = pl.pallas_call(
        _project,
        in_specs=[pl.BlockSpec((16, N), lambda: (0, 0))],
        out_specs=[pl.BlockSpec((8, NP), lambda: (0, 0)),
                   pl.BlockSpec((H // RB, 8), lambda: (0, 0))],
        out_shape=[jax.ShapeDtypeStruct((8, NP), jnp.float32),
                   jax.ShapeDtypeStruct((H // RB, 8), jnp.int32)],
    )(pt)
    proj = proj_t.T

    img = pl.pallas_call(
        _raster,
        grid=(H // RB,),
        in_specs=[
            pl.BlockSpec(memory_space=pltpu.SMEM),
            pl.BlockSpec((NP, 8), lambda i: (0, 0)),
        ],
        out_specs=pl.BlockSpec((3, RB, W), lambda i: (0, i, 0)),
        out_shape=jax.ShapeDtypeStruct((3, H, W), jnp.float32),
        scratch_shapes=[pltpu.VMEM((RB * 2 * 3, GB, 128), jnp.float32)],
        compiler_params=pltpu.CompilerParams(dimension_semantics=("parallel",)),
    )(binfo, proj)
    return img[None]
